# baseline (speedup 1.0000x reference)
import jax
import jax.numpy as jnp
from jax.experimental import pallas as pl

INTERPRET = False

_TB = 512
_CHUNK = 2048


def _dist_argmin_kernel(z_ref, cb_ref, z2_ref, c2_ref, idx_ref, loss_ref):
    pid = pl.program_id(0)
    tb, d = z_ref.shape
    kc = cb_ref.shape[0]
    zb2 = z_ref[...] * 2.0
    z2 = z2_ref[...]
    nchunks = kc // _CHUNK

    def body(j, carry):
        bval, bidx = carry
        cb_chunk = cb_ref[pl.ds(j * _CHUNK, _CHUNK), :]
        c2_chunk = c2_ref[:, pl.ds(j * _CHUNK, _CHUNK)]
        mm2 = jax.lax.dot_general(
            zb2, cb_chunk, (((1,), (1,)), ((), ())),
            preferred_element_type=jnp.float32)
        dist = (z2 + c2_chunk) - mm2
        cval = jnp.min(dist, axis=1)
        iota = jax.lax.broadcasted_iota(jnp.int32, (tb, _CHUNK), 1).astype(
            jnp.float32)
        cidx = jnp.min(
            jnp.where(dist == cval[:, None], iota, jnp.float32(2 ** 24)),
            axis=1) + jnp.float32(j * _CHUNK)
        upd = cval < bval
        return jnp.where(upd, cval, bval), jnp.where(upd, cidx, bidx)

    init = (jnp.full((tb,), jnp.inf, jnp.float32),
            jnp.zeros((tb,), jnp.float32))
    bval, bidx = jax.lax.fori_loop(0, nchunks, body, init)
    idx_ref[...] = bidx.astype(jnp.int32)

    @pl.when(pid == 0)
    def _():
        loss_ref[...] = jnp.zeros((1, 1), jnp.float32)

    loss_ref[...] += jnp.sum(bval).reshape(1, 1)

    @pl.when(pid == pl.num_programs(0) - 1)
    def _():
        ntok_total = pl.num_programs(0) * tb
        m = loss_ref[...] / jnp.float32(ntok_total * d)
        loss_ref[...] = m + 0.25 * m


def kernel(z, codebook):
    b, l, d = z.shape
    kc = codebook.shape[0]
    ntok = b * l
    flat_z = z.reshape(-1, d)
    z2 = jnp.sum(flat_z ** 2, axis=-1, keepdims=True)
    c2 = jnp.sum(codebook ** 2, axis=-1, keepdims=True).T
    idx_flat, loss = pl.pallas_call(
        _dist_argmin_kernel,
        grid=(ntok // _TB,),
        in_specs=[
            pl.BlockSpec((_TB, d), lambda i: (i, 0)),
            pl.BlockSpec((kc, d), lambda i: (0, 0)),
            pl.BlockSpec((_TB, 1), lambda i: (i, 0)),
            pl.BlockSpec((1, kc), lambda i: (0, 0)),
        ],
        out_specs=[
            pl.BlockSpec((_TB,), lambda i: (i,)),
            pl.BlockSpec((1, 1), lambda i: (0, 0)),
        ],
        out_shape=[
            jax.ShapeDtypeStruct((ntok,), jnp.int32),
            jax.ShapeDtypeStruct((1, 1), jnp.float32),
        ],
        interpret=INTERPRET,
    )(flat_z, codebook, z2, c2)
    quantized = jnp.take(codebook, idx_flat, axis=0)
    qst = flat_z + (quantized - flat_z)
    return qst.reshape(b, l, d), idx_flat.reshape(b, l), loss[0, 0]
